# baseline (speedup 1.0000x reference)
import functools

import jax
import jax.numpy as jnp
from jax import lax
from jax.experimental import pallas as pl
from jax.experimental.pallas import tpu as pltpu
from jax.experimental.pallas import tpu_sc as plsc

VOCAB = 100000
DIM = 128
SEQ = 50
SEQ_PAD = 56
NC = 2
NS = 16
NW = NC * NS
CHUNK = 128
NCHUNK = (4096 // NW) * SEQ_PAD // CHUNK
NBUF = 7
NGROUP = NCHUNK // NBUF


def _body(x_hbm, tbl_hbm, out_hbm, idx_v, rows_v, gsem, osem):
    wid = lax.axis_index("s") * NC + lax.axis_index("c")
    pltpu.sync_copy(x_hbm.at[wid], idx_v)

    def start_gather(j, b):
        pltpu.async_copy(tbl_hbm.at[idx_v.at[j]], rows_v.at[b], gsem.at[b])

    def wait_gather(b):
        pltpu.make_async_copy(
            tbl_hbm.at[idx_v.at[0]], rows_v.at[b], gsem.at[b]).wait()

    def start_out(j, b):
        pltpu.async_copy(rows_v.at[b], out_hbm.at[wid, j], osem.at[b])

    def wait_out(b):
        pltpu.make_async_copy(
            rows_v.at[b], out_hbm.at[wid, 0], osem.at[b]).wait()

    for b in range(NBUF):
        start_gather(b, b)

    def group(g, carry):
        for b in range(NBUF):
            wait_gather(b)
            start_out(g * NBUF + b, b)
        for b in range(NBUF):
            wait_out(b)

            @pl.when(g + 1 < NGROUP)
            def _():
                start_gather((g + 1) * NBUF + b, b)

        return carry

    lax.fori_loop(0, NGROUP, group, 0)


def _trim_body(in_ref, out_ref):
    out_ref[...] = in_ref[:, :SEQ, :]


BB = 16


@jax.jit
def _run(x_pad, emb_weight):
    mesh = plsc.VectorSubcoreMesh(core_axis_name="c", subcore_axis_name="s")
    gather_k = pl.kernel(
        _body,
        out_type=jax.ShapeDtypeStruct((NW, NCHUNK, CHUNK, DIM), jnp.float32),
        mesh=mesh,
        scratch_types=[
            pltpu.VMEM((NCHUNK, CHUNK), jnp.int32),
            pltpu.VMEM((NBUF, CHUNK, DIM), jnp.float32),
            pltpu.SemaphoreType.DMA((NBUF,)),
            pltpu.SemaphoreType.DMA((NBUF,)),
        ],
    )
    staged = gather_k(x_pad, emb_weight).reshape(4096, SEQ_PAD, DIM)
    return pl.pallas_call(
        _trim_body,
        out_shape=jax.ShapeDtypeStruct((4096, SEQ, DIM), jnp.float32),
        grid=(4096 // BB,),
        in_specs=[pl.BlockSpec((BB, SEQ_PAD, DIM), lambda i: (i, 0, 0))],
        out_specs=pl.BlockSpec((BB, SEQ, DIM), lambda i: (i, 0, 0)),
    )(staged)


def kernel(x, emb_weight):
    b, s = x.shape
    x_pad = jnp.pad(x.astype(jnp.int32), ((0, 0), (0, SEQ_PAD - s)))
    x_pad = x_pad.reshape(NW, NCHUNK, CHUNK)
    return _run(x_pad, emb_weight)


# speedup vs baseline: 4.0809x; 4.0809x over previous
import functools

import jax
import jax.numpy as jnp
from jax import lax
from jax.experimental import pallas as pl
from jax.experimental.pallas import tpu as pltpu
from jax.experimental.pallas import tpu_sc as plsc

VOCAB = 100000
DIM = 128
SEQ = 50
SEQ_PAD = 56
NC = 2
NS = 16
NW = NC * NS
CHUNK = 128
NCHUNK = (4096 // NW) * SEQ_PAD // CHUNK
NBUF = 7
NGROUP = NCHUNK // NBUF


def _body(x_hbm, tbl_hbm, out_hbm, idx_v, rows_v, gsem, osem):
    wid = lax.axis_index("s") * NC + lax.axis_index("c")
    pltpu.sync_copy(x_hbm.at[wid], idx_v)

    def start_gather(j, b):
        pltpu.async_copy(tbl_hbm.at[idx_v.at[j]], rows_v.at[b], gsem.at[b])

    def wait_gather(b):
        pltpu.make_async_copy(
            tbl_hbm.at[idx_v.at[0]], rows_v.at[b], gsem.at[b]).wait()

    def start_out(j, b):
        pltpu.async_copy(rows_v.at[b], out_hbm.at[wid, j], osem.at[b])

    def wait_out(b):
        pltpu.make_async_copy(
            rows_v.at[b], out_hbm.at[wid, 0], osem.at[b]).wait()

    for b in range(NBUF):
        start_gather(b, b)

    def group(g, carry):
        for b in range(NBUF):
            wait_gather(b)
            start_out(g * NBUF + b, b)
        for b in range(NBUF):
            wait_out(b)

            @pl.when(g + 1 < NGROUP)
            def _():
                start_gather((g + 1) * NBUF + b, b)

        return carry

    lax.fori_loop(0, NGROUP, group, 0)


def _trim_body(in_ref, out_ref):
    out_ref[...] = in_ref[:, :SEQ, :]


BB = 16


@jax.jit
def _run(x_pad, emb_weight):
    mesh = plsc.VectorSubcoreMesh(core_axis_name="c", subcore_axis_name="s")
    gather_k = pl.kernel(
        _body,
        out_type=jax.ShapeDtypeStruct((NW, NCHUNK, CHUNK, DIM), jnp.float32),
        mesh=mesh,
        scratch_types=[
            pltpu.VMEM((NCHUNK, CHUNK), jnp.int32),
            pltpu.VMEM((NBUF, CHUNK, DIM), jnp.float32),
            pltpu.SemaphoreType.DMA((NBUF,)),
            pltpu.SemaphoreType.DMA((NBUF,)),
        ],
    )
    staged = gather_k(x_pad, emb_weight).reshape(4096, SEQ_PAD, DIM)
    return pl.pallas_call(
        _trim_body,
        out_shape=jax.ShapeDtypeStruct((4096, SEQ, DIM), jnp.float32),
        grid=(4096 // BB,),
        in_specs=[pl.BlockSpec((BB, SEQ_PAD, DIM), lambda i: (i, 0, 0))],
        out_specs=pl.BlockSpec((BB, SEQ, DIM), lambda i: (i, 0, 0)),
    )(staged)


def kernel(x, emb_weight):
    b, s = x.shape
    pad = (jnp.arange(b * (SEQ_PAD - s), dtype=jnp.int32) * 2711) % VOCAB
    x_pad = jnp.concatenate(
        [x.astype(jnp.int32), pad.reshape(b, SEQ_PAD - s)], axis=1)
    x_pad = x_pad.reshape(NW, NCHUNK, CHUNK)
    return _run(x_pad, emb_weight)


# speedup vs baseline: 8.4912x; 2.0807x over previous
import functools

import jax
import jax.numpy as jnp
from jax import lax
from jax.experimental import pallas as pl
from jax.experimental.pallas import tpu as pltpu
from jax.experimental.pallas import tpu_sc as plsc

VOCAB = 100000
DIM = 128
SEQ = 50
SEQ_PAD = 56
NC = 2
NS = 16
NW = NC * NS
BPW = 4096 // NW
NBUF = 8
NGROUP = BPW // NBUF


def _body(x_hbm, tbl_hbm, out_hbm, idx_v, rows_v, gsem, osem):
    wid = lax.axis_index("s") * NC + lax.axis_index("c")
    pltpu.sync_copy(x_hbm.at[wid], idx_v)

    def start_gather(b, buf):
        pltpu.async_copy(
            tbl_hbm.at[idx_v.at[b, pl.ds(0, SEQ_PAD)]], rows_v.at[buf],
            gsem.at[buf])

    def wait_gather(buf):
        pltpu.make_async_copy(
            tbl_hbm.at[idx_v.at[0, pl.ds(0, SEQ_PAD)]], rows_v.at[buf],
            gsem.at[buf]).wait()

    def start_out(b, buf):
        pltpu.async_copy(
            rows_v.at[buf, pl.ds(0, SEQ)], out_hbm.at[wid * BPW + b],
            osem.at[buf])

    def wait_out(buf):
        pltpu.make_async_copy(
            rows_v.at[buf, pl.ds(0, SEQ)], out_hbm.at[0], osem.at[buf]).wait()

    for buf in range(NBUF):
        start_gather(buf, buf)

    def group(g, carry):
        for buf in range(NBUF):
            wait_gather(buf)
            start_out(g * NBUF + buf, buf)
        for buf in range(NBUF):
            wait_out(buf)

            @pl.when(g + 1 < NGROUP)
            def _():
                start_gather((g + 1) * NBUF + buf, buf)

        return carry

    lax.fori_loop(0, NGROUP, group, 0)


@jax.jit
def _run(x_pad, emb_weight):
    mesh = plsc.VectorSubcoreMesh(core_axis_name="c", subcore_axis_name="s")
    k = pl.kernel(
        _body,
        out_type=jax.ShapeDtypeStruct((4096, SEQ, DIM), jnp.float32),
        mesh=mesh,
        scratch_types=[
            pltpu.VMEM((BPW, 128), jnp.int32),
            pltpu.VMEM((NBUF, SEQ_PAD, DIM), jnp.float32),
            pltpu.SemaphoreType.DMA((NBUF,)),
            pltpu.SemaphoreType.DMA((NBUF,)),
        ],
        compiler_params=pltpu.CompilerParams(use_tc_tiling_on_sc=True),
    )
    return k(x_pad, emb_weight)


def kernel(x, emb_weight):
    b, s = x.shape
    pad = (jnp.arange(b * (128 - s), dtype=jnp.int32) * 2711) % VOCAB
    x_pad = jnp.concatenate(
        [x.astype(jnp.int32), pad.reshape(b, 128 - s)], axis=1)
    x_pad = x_pad.reshape(NW, BPW, 128)
    return _run(x_pad, emb_weight)


# speedup vs baseline: 8.7407x; 1.0294x over previous
import functools

import jax
import jax.numpy as jnp
from jax import lax
from jax.experimental import pallas as pl
from jax.experimental.pallas import tpu as pltpu
from jax.experimental.pallas import tpu_sc as plsc

VOCAB = 100000
DIM = 128
SEQ = 50
SEQ_PAD = 56
NC = 2
NS = 16
NW = NC * NS
BPW = 4096 // NW
NBUF = 8
NGROUP = BPW // NBUF


def _body(x_hbm, tbl_hbm, out_hbm, idx_v, rows_v, gsem, osem):
    wid = lax.axis_index("s") * NC + lax.axis_index("c")
    pltpu.sync_copy(x_hbm.at[wid], idx_v)

    def start_gather(b, buf):
        pltpu.async_copy(
            tbl_hbm.at[idx_v.at[b, pl.ds(0, SEQ)]],
            rows_v.at[buf, pl.ds(0, SEQ)], gsem.at[buf])

    def wait_gather(buf):
        pltpu.make_async_copy(
            tbl_hbm.at[idx_v.at[0, pl.ds(0, SEQ)]],
            rows_v.at[buf, pl.ds(0, SEQ)], gsem.at[buf]).wait()

    def start_out(b, buf):
        pltpu.async_copy(
            rows_v.at[buf, pl.ds(0, SEQ)], out_hbm.at[wid * BPW + b],
            osem.at[buf])

    def wait_out(buf):
        pltpu.make_async_copy(
            rows_v.at[buf, pl.ds(0, SEQ)], out_hbm.at[0], osem.at[buf]).wait()

    for buf in range(NBUF):
        start_gather(buf, buf)

    def group(g, carry):
        for buf in range(NBUF):
            wait_gather(buf)
            start_out(g * NBUF + buf, buf)
        for buf in range(NBUF):
            wait_out(buf)

            @pl.when(g + 1 < NGROUP)
            def _():
                start_gather((g + 1) * NBUF + buf, buf)

        return carry

    lax.fori_loop(0, NGROUP, group, 0)


@jax.jit
def _run(x_r, emb_weight):
    mesh = plsc.VectorSubcoreMesh(core_axis_name="c", subcore_axis_name="s")
    k = pl.kernel(
        _body,
        out_type=jax.ShapeDtypeStruct((4096, SEQ, DIM), jnp.float32),
        mesh=mesh,
        scratch_types=[
            pltpu.VMEM((BPW, SEQ), jnp.int32),
            pltpu.VMEM((NBUF, SEQ_PAD, DIM), jnp.float32),
            pltpu.SemaphoreType.DMA((NBUF,)),
            pltpu.SemaphoreType.DMA((NBUF,)),
        ],
        compiler_params=pltpu.CompilerParams(use_tc_tiling_on_sc=True),
    )
    return k(x_r, emb_weight)


def kernel(x, emb_weight):
    b, s = x.shape
    x_r = x.astype(jnp.int32).reshape(NW, BPW, s)
    return _run(x_r, emb_weight)
